# baseline (speedup 1.0000x reference)
import functools

import jax
import jax.numpy as jnp
from jax import lax
from jax.experimental import pallas as pl
from jax.experimental.pallas import tpu as pltpu
from jax.experimental.pallas import tpu_sc as plsc

NC, NS, L = 2, 16, 16
NW = NC * NS
B = 16384
E = 64
NROWS = 1000000
CH = 16384
CSH = 14
NFULL = NROWS // CH
TAIL = NROWS - NFULL * CH
RING = 384
OUTROWS = B + 128

_mesh = plsc.VectorSubcoreMesh(core_axis_name="c", subcore_axis_name="s")


@functools.partial(
    pl.kernel,
    out_type=(
        jax.ShapeDtypeStruct((OUTROWS, 128), jnp.float32),
        jax.ShapeDtypeStruct((OUTROWS, 128), jnp.float32),
    ),
    mesh=_mesh,
    scratch_types=[
        pltpu.VMEM((B,), jnp.int32),
        pltpu.VMEM((B + L,), jnp.int32),
        pltpu.VMEM((CH,), jnp.float32),
        pltpu.VMEM((CH,), jnp.float32),
        pltpu.VMEM((RING, 128), jnp.float32),
        pltpu.VMEM((RING // 128, 128), jnp.int32),
        pltpu.SemaphoreType.DMA,
        pltpu.SemaphoreType.DMA,
        pltpu.SemaphoreType.DMA,
    ],
    compiler_params=pltpu.CompilerParams(
        use_tc_tiling_on_sc=True, needs_layout_passes=False),
)
def _sc_scan_gather(uid, bid, ut, bt, ub, bb, u_out, b_out,
                    ids_v, ma_v, slab_a, slab_b, ring_v, ridx_v,
                    sem, sem_a, sem_b):
    w = lax.axis_index("s") * NC + lax.axis_index("c")
    lanes = lax.iota(jnp.int32, L)
    padidx = jnp.full((L,), B, jnp.int32)
    zeros16 = jnp.zeros((L,), jnp.int32)

    def run_table(idhbm, tab, bias1d, out):
        pltpu.sync_copy(idhbm, ids_v)

        def find_matches(chunk):
            def pa(c, nm):
                v = ids_v[pl.ds(c * L, L)]
                m = lax.shift_right_logical(v, CSH) == chunk
                packed = ((v & (CH - 1)) << CSH) | (c * L + lanes)
                plsc.store_compressed(ma_v.at[pl.ds(nm, L)], packed, mask=m)
                return nm + jnp.max(plsc.all_reduce_population_count(m))

            nm = lax.fori_loop(0, B // L, pa, jnp.int32(0))
            ma_v[pl.ds(nm, L)] = zeros16
            return nm

        def process_chunk(chunk, csz):
            nm = find_matches(chunk)
            cbase = chunk * CH

            def start(g, buf, bsem):
                def tabcopy(_):
                    base = pl.multiple_of(g * NROWS + cbase, 64)
                    pltpu.async_copy(tab.at[pl.ds(base, csz)],
                                     buf.at[pl.ds(0, csz)], bsem)
                    return jnp.int32(0)

                def biascopy(_):
                    pltpu.async_copy(bias1d.at[pl.ds(cbase, csz)],
                                     buf.at[pl.ds(0, csz)], bsem)
                    return jnp.int32(0)

                lax.cond(g < E, tabcopy, lambda _: jnp.int32(0), 0)
                lax.cond(g == E, biascopy, lambda _: jnp.int32(0), 0)

            def wait(g, buf, bsem):
                def go(_):
                    pltpu.make_async_copy(
                        tab.at[pl.ds(0, csz)], buf.at[pl.ds(0, csz)],
                        bsem).wait()
                    return jnp.int32(0)

                lax.cond(g <= E, go, lambda _: jnp.int32(0), 0)

            def batch_body(b):
                boff = b * RING
                cnt = jnp.minimum(nm - boff, RING)
                ngrp = lax.shift_right_logical(cnt + (L - 1), 4)

                for g in range(RING // 128):
                    for q in range(128 // L):
                        ridx_v[g, pl.ds(q * L, L)] = padidx

                def fill_ridx(j, _):
                    mv = ma_v[pl.ds(boff + j * L, L)]
                    p = j * L + lanes
                    plsc.store_scatter(
                        ridx_v,
                        [lax.shift_right_logical(p, 7), p & 127],
                        mv & (CH - 1), mask=p < cnt)
                    return _

                lax.fori_loop(0, ngrp, fill_ridx, jnp.int32(0))

                bufs = ((slab_a, sem_a), (slab_b, sem_b))
                start(jnp.int32(0), *bufs[0])
                start(jnp.int32(1), *bufs[1])

                def proc(f, buf):
                    def pg(j, _):
                        mv = ma_v[pl.ds(boff + j * L, L)]
                        col = lax.shift_right_logical(mv, CSH)
                        gv = plsc.load_gather(buf, [col])
                        plsc.store_scatter(
                            ring_v, [j * L + lanes,
                                     jnp.broadcast_to(f, (L,))], gv)
                        return _

                    lax.fori_loop(0, ngrp, pg, jnp.int32(0))

                def f_body(f2, _):
                    for ph in range(2):
                        f = 2 * f2 + ph
                        wait(f, *bufs[ph])
                        lax.cond(f <= E,
                                 lambda fv: (proc(fv, bufs[ph][0]),
                                             jnp.int32(0))[1],
                                 lambda fv: jnp.int32(0), f)
                        start(f + 2, *bufs[ph])
                    return _

                lax.fori_loop(0, (E + 2) // 2, f_body, jnp.int32(0))

                for g in range(RING // 128):
                    def flush(_):
                        pltpu.async_copy(
                            ring_v.at[pl.ds(g * 128, 128)],
                            out.at[ridx_v.at[g]], sem).wait()
                        return jnp.int32(0)

                    lax.cond(cnt > g * 128, flush,
                             lambda _: jnp.int32(0), 0)

            def batch_cond(b):
                return b * RING < nm

            lax.while_loop(batch_cond,
                           lambda b: (batch_body(b), b + 1)[1],
                           jnp.int32(0))

        process_chunk(w, CH)
        process_chunk(jnp.where(w + NW < NFULL, w + NW, jnp.int32(99)), CH)
        def do_tail(_):
            process_chunk(jnp.int32(NFULL), TAIL)
            return jnp.int32(0)

        lax.cond(w == NW - 1, do_tail, lambda _: jnp.int32(0), 0)

    run_table(uid, ut, ub, u_out)
    run_table(bid, bt, bb, b_out)


def _epilogue(u_ref, b_ref, out_ref):
    u = u_ref[pl.ds(0, B), pl.ds(0, E)]
    bk = b_ref[pl.ds(0, B), pl.ds(0, E)]
    s = jnp.sum(u * bk)
    ubias = u_ref[pl.ds(0, B), pl.ds(E, 1)]
    bbias = b_ref[pl.ds(0, B), pl.ds(E, 1)]
    out_ref[...] = jax.nn.sigmoid(s + ubias + bbias)


def kernel(inputs, user_table, user_bias_table, book_table, book_bias_table):
    uid = inputs[:, 0].reshape(-1)
    bid = inputs[:, 1].reshape(-1)
    u_out, b_out = _sc_scan_gather(
        uid, bid, user_table.T.reshape(-1), book_table.T.reshape(-1),
        user_bias_table.reshape(-1), book_bias_table.reshape(-1))
    return pl.pallas_call(
        _epilogue,
        out_shape=jax.ShapeDtypeStruct((B, 1), jnp.float32),
    )(u_out, b_out)


# speedup vs baseline: 6.7297x; 6.7297x over previous
import functools

import jax
import jax.numpy as jnp
from jax import lax
from jax.experimental import pallas as pl
from jax.experimental.pallas import tpu as pltpu
from jax.experimental.pallas import tpu_sc as plsc

NC, NS, L = 2, 16, 16
NW = NC * NS
B = 16384
E = 64
NROWS = 1000000
WIN = 2048
WSH = 11
NWIN = NROWS // WIN
TAILB = NWIN * WIN
TAILN = NROWS - TAILB
TMAX = NWIN // NW
BSZ = 256
RING = 384
OUTROWS = B + 128

_mesh = plsc.VectorSubcoreMesh(core_axis_name="c", subcore_axis_name="s")


@functools.partial(
    pl.kernel,
    out_type=(
        jax.ShapeDtypeStruct((OUTROWS, 128), jnp.float32),
        jax.ShapeDtypeStruct((OUTROWS, 128), jnp.float32),
    ),
    mesh=_mesh,
    scratch_types=[
        pltpu.VMEM((B + L,), jnp.int32),
        pltpu.VMEM((L,), jnp.int32),
        pltpu.VMEM((L,), jnp.int32),
        pltpu.VMEM((B + L,), jnp.int32),
        pltpu.VMEM((8, WIN), jnp.float32),
        pltpu.VMEM((8, WIN), jnp.float32),
        pltpu.VMEM((WIN,), jnp.float32),
        pltpu.VMEM((E, 64), jnp.float32),
        pltpu.VMEM((64,), jnp.float32),
        pltpu.VMEM((RING, 128), jnp.float32),
        pltpu.VMEM((RING // 128, 128), jnp.int32),
        pltpu.SemaphoreType.DMA,
        pltpu.SemaphoreType.DMA,
        pltpu.SemaphoreType.DMA,
    ],
    compiler_params=pltpu.CompilerParams(
        use_tc_tiling_on_sc=True, needs_layout_passes=False),
)
def _sc_scan_gather(uid, bid, ut, bt, ub, bb, u_out, b_out,
                    ids_v, cnt_v, off_v, ma_v, slab_a, slab_b, bias_v,
                    tail_v, tailb_v, ring_v, ridx_v, sem, sem_a, sem_b):
    w = lax.axis_index("s") * NC + lax.axis_index("c")
    lanes = lax.iota(jnp.int32, L)
    lane0 = lanes == 0
    padidx = jnp.full((L,), B, jnp.int32)
    bufs = ((slab_a, sem_a), (slab_b, sem_b))

    def reset_ridx():
        for g in range(RING // 128):
            for q in range(128 // L):
                ridx_v[g, pl.ds(q * L, L)] = padidx

    def run_table(idhbm, tab, bias1d, out):
        pltpu.sync_copy(idhbm, ids_v.at[pl.ds(0, B)])
        reset_ridx()

        cnt_v[pl.ds(0, L)] = jnp.zeros((L,), jnp.int32)
        ones16 = jnp.full((L,), 1, jnp.int32)

        def pa(c, nm):
            v = ids_v[pl.ds(c * L, L)]
            win = lax.shift_right_logical(v, WSH)
            m = (win & (NW - 1)) == w
            slot = lax.shift_right_logical(win, 5)
            packed = (slot << 25) | ((v & (WIN - 1)) << 14) | (c * L + lanes)
            plsc.store_compressed(ma_v.at[pl.ds(nm, L)], packed, mask=m)
            plsc.addupdate_scatter(cnt_v, [slot], ones16, mask=m)
            return nm + jnp.max(plsc.all_reduce_population_count(m))

        nm = lax.fori_loop(0, B // L, pa, jnp.int32(0))

        c16 = cnt_v[pl.ds(0, L)]
        inc = plsc.cumsum(c16)
        off_v[pl.ds(0, L)] = inc - c16

        def place(m, _):
            p = ma_v[pl.ds(m, L)][0]
            slot = lax.shift_right_logical(p, 25)
            cur = off_v[pl.ds(slot, L)][0]
            plsc.store_scatter(ids_v, [jnp.broadcast_to(cur, (L,))],
                               jnp.broadcast_to(p, (L,)), mask=lane0)
            plsc.store_scatter(off_v, [jnp.broadcast_to(slot, (L,))],
                               jnp.broadcast_to(cur + 1, (L,)), mask=lane0)
            return _

        lax.fori_loop(0, nm, place, jnp.int32(0))
        ids_v[pl.ds(nm, L)] = jnp.zeros((L,), jnp.int32)

        def flush(rfill):
            def go(_):
                for g in range(RING // 128):
                    def fl(_):
                        pltpu.async_copy(
                            ring_v.at[pl.ds(g * 128, 128)],
                            out.at[ridx_v.at[g]], sem).wait()
                        return jnp.int32(0)

                    lax.cond(rfill > g * 128, fl, lambda _: jnp.int32(0), 0)
                reset_ridx()
                return jnp.int32(0)

            lax.cond(rfill > 0, go, lambda _: jnp.int32(0), 0)

        def proc_batch(mstart, bcnt, rfill, cbase, csz, coff):
            ngrp = lax.shift_right_logical(bcnt + (L - 1), 4)

            def start(ss, buf, bsem):
                pltpu.async_copy(
                    tab.at[pl.ds(pl.multiple_of(ss * 8, 8), 8),
                           pl.ds(pl.multiple_of(cbase, 512), csz)],
                    buf.at[:, pl.ds(0, csz)], bsem)

            def wait(buf, bsem):
                pltpu.make_async_copy(
                    tab.at[pl.ds(0, 8), pl.ds(0, csz)],
                    buf.at[:, pl.ds(0, csz)], bsem).wait()

            pltpu.sync_copy(
                bias1d.at[pl.ds(pl.multiple_of(cbase, 512), csz)],
                bias_v.at[pl.ds(0, csz)])
            start(0, *bufs[0])
            start(1, *bufs[1])

            for ss in range(8):
                buf, bsem = bufs[ss % 2]
                wait(buf, bsem)

                def pg(j, _):
                    mv = ids_v[pl.ds(mstart + j * L, L)]
                    col = (lax.shift_right_logical(mv, 14) & (WIN - 1)) - coff
                    rows = rfill + j * L + lanes
                    for fr in range(8):
                        gv = plsc.load_gather(
                            buf, [jnp.full((L,), fr, jnp.int32), col])
                        plsc.store_scatter(
                            ring_v,
                            [rows, jnp.full((L,), ss * 8 + fr, jnp.int32)],
                            gv)
                    return _

                lax.fori_loop(0, ngrp, pg, jnp.int32(0))
                if ss < 6:
                    start(ss + 2, buf, bsem)

            def pbias(j, _):
                mv = ids_v[pl.ds(mstart + j * L, L)]
                col = (lax.shift_right_logical(mv, 14) & (WIN - 1)) - coff
                rows = rfill + j * L + lanes
                gv = plsc.load_gather(bias_v, [col])
                plsc.store_scatter(
                    ring_v, [rows, jnp.full((L,), E, jnp.int32)], gv)
                samp = mv & (B - 1)
                p = rfill + j * L + lanes
                plsc.store_scatter(
                    ridx_v, [lax.shift_right_logical(p, 7), p & 127],
                    samp, mask=(j * L + lanes) < bcnt)
                return _

            lax.fori_loop(0, ngrp, pbias, jnp.int32(0))
            return rfill + bcnt

        def window_body(t, rfill):
            def go(rfill):
                end = off_v[pl.ds(t, L)][0]
                cnt = cnt_v[pl.ds(t, L)][0]
                mstart = end - cnt
                cbase = (w + NW * t) * WIN

                def bcond(carry):
                    b, _ = carry
                    return b * BSZ < cnt

                def bbody(carry):
                    b, rfill = carry
                    bcnt = jnp.minimum(cnt - b * BSZ, BSZ)

                    def fl(r):
                        flush(r)
                        return jnp.int32(0)

                    rfill = lax.cond(
                        rfill + BSZ > RING, fl, lambda r: r, rfill)
                    rfill = proc_batch(mstart + b * BSZ, bcnt, rfill,
                                       cbase, WIN, 0)
                    return b + 1, rfill

                return lax.while_loop(bcond, bbody, (jnp.int32(0),
                                                     rfill))[1]

            valid = w + NW * t < NWIN
            return lax.cond(valid, go, lambda r: r, rfill)

        rfill = lax.fori_loop(0, TMAX + 1, window_body, jnp.int32(0))

        def tail_matches(cstart, cw):
            def pa(c, nmt):
                v = ma_v[pl.ds(c * L, L)]
                col = v - cstart
                m = (col >= 0) & (col < cw)
                packed = (col << 14) | (c * L + lanes)
                plsc.store_compressed(ids_v.at[pl.ds(nmt, L)], packed,
                                      mask=m)
                return nmt + jnp.max(plsc.all_reduce_population_count(m))

            nmt = lax.fori_loop(0, B // L, pa, jnp.int32(0))
            ids_v[pl.ds(nmt, L)] = jnp.zeros((L,), jnp.int32)
            return nmt

        def do_tail(rfill):
            pltpu.sync_copy(idhbm, ma_v.at[pl.ds(0, B)])

            nmt = tail_matches(TAILB, 512)

            def bcond(carry):
                b, _ = carry
                return b * BSZ < nmt

            def bbody(carry):
                b, rfill = carry
                bcnt = jnp.minimum(nmt - b * BSZ, BSZ)

                def fl(r):
                    flush(r)
                    return jnp.int32(0)

                rfill = lax.cond(rfill + BSZ > RING, fl, lambda r: r, rfill)
                rfill = proc_batch(b * BSZ, bcnt, rfill, TAILB, 512, 0)
                return b + 1, rfill

            rfill = lax.while_loop(bcond, bbody, (jnp.int32(0), rfill))[1]

            pltpu.sync_copy(tab.at[:, pl.ds(TAILB + 512, 64)], tail_v)
            pltpu.sync_copy(bias1d.at[pl.ds(TAILB + 512, 64)], tailb_v)
            nmt = tail_matches(TAILB + 512, 64)

            def b2body(carry):
                b, rfill = carry
                bcnt = jnp.minimum(nmt - b * BSZ, BSZ)

                def fl(r):
                    flush(r)
                    return jnp.int32(0)

                rfill = lax.cond(rfill + BSZ > RING, fl, lambda r: r, rfill)
                ngrp = lax.shift_right_logical(bcnt + (L - 1), 4)

                def pg(j, _):
                    mv = ids_v[pl.ds(b * BSZ + j * L, L)]
                    col = lax.shift_right_logical(mv, 14) & (WIN - 1)
                    rows = rfill + j * L + lanes
                    for g in range(E // L):
                        for fr in range(L):
                            f = g * L + fr
                            gv = plsc.load_gather(
                                tail_v, [jnp.full((L,), f, jnp.int32), col])
                            plsc.store_scatter(
                                ring_v, [rows, jnp.full((L,), f, jnp.int32)],
                                gv)
                    gv = plsc.load_gather(tailb_v, [col])
                    plsc.store_scatter(
                        ring_v, [rows, jnp.full((L,), E, jnp.int32)], gv)
                    samp = mv & (B - 1)
                    plsc.store_scatter(
                        ridx_v, [lax.shift_right_logical(rows, 7),
                                 rows & 127],
                        samp, mask=(j * L + lanes) < bcnt)
                    return _

                lax.fori_loop(0, ngrp, pg, jnp.int32(0))
                return b + 1, rfill + bcnt

            rfill = lax.while_loop(bcond, b2body, (jnp.int32(0), rfill))[1]
            return rfill

        rfill = lax.cond(w == TAILB // WIN % NW, do_tail,
                         lambda r: r, rfill)
        flush(rfill)

    run_table(uid, ut, ub, u_out)
    run_table(bid, bt, bb, b_out)


def _epilogue(u_ref, b_ref, out_ref):
    u = u_ref[pl.ds(0, B), pl.ds(0, E)]
    bk = b_ref[pl.ds(0, B), pl.ds(0, E)]
    s = jnp.sum(u * bk)
    ubias = u_ref[pl.ds(0, B), pl.ds(E, 1)]
    bbias = b_ref[pl.ds(0, B), pl.ds(E, 1)]
    out_ref[...] = jax.nn.sigmoid(s + ubias + bbias)


def kernel(inputs, user_table, user_bias_table, book_table, book_bias_table):
    uid = inputs[:, 0].reshape(-1)
    bid = inputs[:, 1].reshape(-1)
    u_out, b_out = _sc_scan_gather(
        uid, bid, user_table.T, book_table.T,
        user_bias_table.reshape(-1), book_bias_table.reshape(-1))
    return pl.pallas_call(
        _epilogue,
        out_shape=jax.ShapeDtypeStruct((B, 1), jnp.float32),
    )(u_out, b_out)


# speedup vs baseline: 18.6276x; 2.7680x over previous
import functools

import jax
import jax.numpy as jnp
from jax import lax
from jax.experimental import pallas as pl
from jax.experimental.pallas import tpu as pltpu
from jax.experimental.pallas import tpu_sc as plsc

NC, NS, L = 2, 16, 16
NW = NC * NS
B = 16384
E = 64
NROWS = 1000000
WIN = 256
WSH = 8
LASTWIN = NROWS // WIN
TMAX = LASTWIN // NW + 1
OUTROWS = B + 128

_mesh = plsc.VectorSubcoreMesh(core_axis_name="c", subcore_axis_name="s")


@functools.partial(
    pl.kernel,
    out_type=(
        jax.ShapeDtypeStruct((OUTROWS, 128), jnp.float32),
        jax.ShapeDtypeStruct((OUTROWS, 128), jnp.float32),
    ),
    mesh=_mesh,
    scratch_types=[
        pltpu.VMEM((B + L,), jnp.int32),
        pltpu.VMEM((144,), jnp.int32),
        pltpu.VMEM((144,), jnp.int32),
        pltpu.VMEM((B + L,), jnp.int32),
        pltpu.VMEM((2 * L,), jnp.int32),
        pltpu.VMEM((E, WIN), jnp.float32),
        pltpu.VMEM((E, WIN), jnp.float32),
        pltpu.VMEM((E, WIN), jnp.float32),
        pltpu.VMEM((WIN,), jnp.float32),
        pltpu.VMEM((WIN,), jnp.float32),
        pltpu.VMEM((WIN,), jnp.float32),
        pltpu.VMEM((E, 64), jnp.float32),
        pltpu.VMEM((64,), jnp.float32),
        pltpu.VMEM((128, 128), jnp.float32),
        pltpu.VMEM((1, 128), jnp.int32),
        pltpu.SemaphoreType.DMA,
        pltpu.SemaphoreType.DMA,
        pltpu.SemaphoreType.DMA,
        pltpu.SemaphoreType.DMA,
    ],
    compiler_params=pltpu.CompilerParams(
        use_tc_tiling_on_sc=True, needs_layout_passes=False),
)
def _sc_scan_gather(uid, bid, ut, bt, ub, bb, u_out, b_out,
                    ids_v, cnt_v, off_v, ma_v, wm_v, slab_a, slab_b, slab_c,
                    bias_a, bias_b, bias_c, tail_v, tailb_v, ring_v, ridx_v,
                    sem, sem_a, sem_b, sem_c):
    w = lax.axis_index("s") * NC + lax.axis_index("c")
    lanes = lax.iota(jnp.int32, L)
    lane0 = lanes == 0
    padidx = jnp.full((L,), B, jnp.int32)

    def reset_ridx():
        for g in range(128 // L):
            ridx_v[0, pl.ds(g * L, L)] = padidx

    def run_table(idhbm, tab, bias1d, out):
        reset_ridx()

        def valid(t):
            return w + NW * t < LASTWIN

        def start_stage(t, slab_ref, bias_ref, ssem):
            def go(_):
                base = pl.multiple_of((w + NW * t) * WIN, WIN)
                pltpu.async_copy(tab.at[:, pl.ds(base, WIN)], slab_ref, ssem)
                pltpu.async_copy(bias1d.at[pl.ds(base, WIN)], bias_ref, ssem)
                return jnp.int32(0)

            lax.cond(valid(t), go, lambda _: jnp.int32(0), jnp.int32(0))

        def wait_stage(t, slab_ref, bias_ref, ssem):
            def go(_):
                pltpu.make_async_copy(
                    tab.at[:, pl.ds(0, WIN)], slab_ref, ssem).wait()
                pltpu.make_async_copy(
                    bias1d.at[pl.ds(0, WIN)], bias_ref, ssem).wait()
                return jnp.int32(0)

            lax.cond(valid(t), go, lambda _: jnp.int32(0), jnp.int32(0))

        bufs = ((slab_a, bias_a, sem_a), (slab_b, bias_b, sem_b),
                (slab_c, bias_c, sem_c))
        start_stage(0, *bufs[0])
        start_stage(1, *bufs[1])

        pltpu.sync_copy(idhbm, ids_v.at[pl.ds(0, B)])
        zeros16 = jnp.zeros((L,), jnp.int32)
        ones16 = jnp.full((L,), 1, jnp.int32)
        for g in range(144 // L):
            cnt_v[pl.ds(g * L, L)] = zeros16

        def pa(c, nm):
            v = ids_v[pl.ds(c * L, L)]
            win = lax.shift_right_logical(v, WSH)
            m = (win & 31) == w
            slot = lax.shift_right_logical(win, 5)
            packed = (slot << 22) | ((v & (WIN - 1)) << 14) | (c * L + lanes)
            plsc.store_compressed(ma_v.at[pl.ds(nm, L)], packed, mask=m)
            plsc.addupdate_scatter(cnt_v, [slot], ones16, mask=m)
            return nm + jnp.max(plsc.all_reduce_population_count(m))

        nm = lax.fori_loop(0, B // L, pa, jnp.int32(0))

        carry = jnp.int32(0)
        for g in range(TMAX // L + 1):
            c16 = cnt_v[pl.ds(g * L, L)]
            inc = plsc.cumsum(c16) + carry
            off_v[pl.ds(g * L, L)] = inc - c16
            carry = inc[L - 1]

        def place(m, _):
            p = ma_v[pl.ds(m, L)][0]
            slot = lax.shift_right_logical(p, 22)
            cur = off_v[pl.ds(slot, L)][0]
            plsc.store_scatter(ids_v, [jnp.broadcast_to(cur, (L,))],
                               jnp.broadcast_to(p, (L,)), mask=lane0)
            plsc.store_scatter(off_v, [jnp.broadcast_to(slot, (L,))],
                               jnp.broadcast_to(cur + 1, (L,)), mask=lane0)
            return _

        lax.fori_loop(0, nm, place, jnp.int32(0))

        def scan_window(t, rfill, slab_ref, bias_ref):
                end = off_v[pl.ds(t, L)][0]
                cnt = cnt_v[pl.ds(t, L)][0]
                start = end - cnt

                def match_body(j, rfill):
                    p = ids_v[pl.ds(start + j, L)][0]
                    col = lax.shift_right_logical(p, 14) & (WIN - 1)
                    k = p & (B - 1)
                    rpos = rfill & 127
                    colv = jnp.broadcast_to(col, (L,))
                    for g in range(E // L):
                        gv = plsc.load_gather(
                            slab_ref, [lanes + g * L, colv])
                        ring_v[rpos, pl.ds(g * L, L)] = gv
                    bv = plsc.load_gather(bias_ref, [colv])
                    ring_v[rpos, pl.ds(E, L)] = bv
                    plsc.store_scatter(
                        ridx_v.at[0], [jnp.broadcast_to(rpos, (L,))],
                        jnp.broadcast_to(k, (L,)), mask=lane0)
                    rfill = rfill + 1

                    def flush(_):
                        pltpu.async_copy(ring_v, out.at[ridx_v.at[0]],
                                         sem).wait()
                        reset_ridx()
                        return jnp.int32(0)

                    lax.cond((rfill & 127) == 0, flush,
                             lambda _: jnp.int32(0), jnp.int32(0))
                    return rfill

                return lax.fori_loop(0, cnt, match_body, rfill)

        def process(t, rfill, slab_ref, bias_ref):
            return lax.cond(
                valid(t),
                lambda r: scan_window(t, r, slab_ref, bias_ref),
                lambda r: r, rfill)

        def tri_body(tp, rfill):
            for ph in range(3):
                t = 3 * tp + ph
                wait_stage(t, *bufs[ph])
                start_stage(t + 2, *bufs[(ph + 2) % 3])
                rfill = process(t, rfill, bufs[ph][0], bufs[ph][1])
            return rfill

        rfill = lax.fori_loop(0, TMAX // 3, tri_body, jnp.int32(0))

        def do_tail(rfill):
            tail = NROWS - LASTWIN * WIN
            pltpu.sync_copy(tab.at[:, pl.ds(LASTWIN * WIN, tail)], tail_v)
            pltpu.sync_copy(bias1d.at[pl.ds(LASTWIN * WIN, tail)], tailb_v)
            return scan_window(jnp.int32(LASTWIN // NW), rfill,
                               tail_v, tailb_v)

        rfill = lax.cond(w == LASTWIN % NW, do_tail, lambda r: r, rfill)

        def drain(_):
            pltpu.async_copy(ring_v, out.at[ridx_v.at[0]], sem).wait()
            return jnp.int32(0)

        lax.cond((rfill & 127) != 0, drain, lambda _: jnp.int32(0),
                 jnp.int32(0))

    run_table(uid, ut, ub, u_out)
    run_table(bid, bt, bb, b_out)


def _epilogue(u_ref, b_ref, out_ref):
    u = u_ref[pl.ds(0, B), pl.ds(0, E)]
    bk = b_ref[pl.ds(0, B), pl.ds(0, E)]
    s = jnp.sum(u * bk)
    ubias = u_ref[pl.ds(0, B), pl.ds(E, 1)]
    bbias = b_ref[pl.ds(0, B), pl.ds(E, 1)]
    out_ref[...] = jax.nn.sigmoid(s + ubias + bbias)


def kernel(inputs, user_table, user_bias_table, book_table, book_bias_table):
    uid = inputs[:, 0].reshape(-1)
    bid = inputs[:, 1].reshape(-1)
    u_out, b_out = _sc_scan_gather(
        uid, bid, user_table.T, book_table.T,
        user_bias_table.reshape(-1), book_bias_table.reshape(-1))
    return pl.pallas_call(
        _epilogue,
        out_shape=jax.ShapeDtypeStruct((B, 1), jnp.float32),
    )(u_out, b_out)
